# baseline (speedup 1.0000x reference)
import jax
import jax.numpy as jnp
from jax.experimental import pallas as pl

BLK = 1024


def _mlp_kernel(inp_ref, we_ref, w1_ref, b1_ref, w2_ref, b2_ref, w3_ref, out_ref):
    x = inp_ref[...]
    e = jnp.maximum(jnp.dot(x, we_ref[...], preferred_element_type=jnp.float32), 0.0)
    h = jnp.maximum(
        jnp.dot(e, w1_ref[...], preferred_element_type=jnp.float32) + b1_ref[...], 0.0
    )
    h = jnp.maximum(
        jnp.dot(h, w2_ref[...], preferred_element_type=jnp.float32) + b2_ref[...], 0.0
    )
    out_ref[...] = jnp.dot(h, w3_ref[...], preferred_element_type=jnp.float32)


def kernel(inp, W_embed, W1, b1, W2, b2, W3):
    B, inp_dim = inp.shape
    c_embed = W_embed.shape[1]
    haz = W1.shape[1]
    half = W2.shape[1]
    out_dim = W3.shape[1]

    b1_2d = b1.reshape(1, haz)
    b2_2d = b2.reshape(1, half)

    grid = (B // BLK,)
    full = lambda i: (0, 0)
    return pl.pallas_call(
        _mlp_kernel,
        grid=grid,
        in_specs=[
            pl.BlockSpec((BLK, inp_dim), lambda i: (i, 0)),
            pl.BlockSpec((inp_dim, c_embed), full),
            pl.BlockSpec((c_embed, haz), full),
            pl.BlockSpec((1, haz), full),
            pl.BlockSpec((haz, half), full),
            pl.BlockSpec((1, half), full),
            pl.BlockSpec((half, out_dim), full),
        ],
        out_specs=pl.BlockSpec((BLK, out_dim), lambda i: (i, 0)),
        out_shape=jax.ShapeDtypeStruct((B, out_dim), jnp.float32),
    )(inp, W_embed, W1, b1_2d, W2, b2_2d, W3)


# speedup vs baseline: 1.0006x; 1.0006x over previous
import jax
import jax.numpy as jnp
from jax import lax
from jax.experimental import pallas as pl
from jax.experimental.pallas import tpu as pltpu

BLK = 1024
_PREC = lax.Precision.DEFAULT


def _dot(a, b):
    return jnp.dot(a, b, preferred_element_type=jnp.float32, precision=_PREC)


def _mlp_kernel(inp_ref, we_ref, w1_ref, b1_ref, w2_ref, b2_ref, w3_ref, out_ref):
    x = inp_ref[...]
    e = jnp.maximum(_dot(x, we_ref[...]), 0.0)
    h = jnp.maximum(_dot(e, w1_ref[...]) + b1_ref[...], 0.0)
    h = jnp.maximum(_dot(h, w2_ref[...]) + b2_ref[...], 0.0)
    out_ref[...] = _dot(h, w3_ref[...])


def kernel(inp, W_embed, W1, b1, W2, b2, W3):
    B, inp_dim = inp.shape
    c_embed = W_embed.shape[1]
    haz = W1.shape[1]
    half = W2.shape[1]
    out_dim = W3.shape[1]

    b1_2d = b1.reshape(1, haz)
    b2_2d = b2.reshape(1, half)

    grid = (B // BLK,)
    full = lambda i: (0, 0)
    return pl.pallas_call(
        _mlp_kernel,
        grid=grid,
        in_specs=[
            pl.BlockSpec((BLK, inp_dim), lambda i: (i, 0)),
            pl.BlockSpec((inp_dim, c_embed), full),
            pl.BlockSpec((c_embed, haz), full),
            pl.BlockSpec((1, haz), full),
            pl.BlockSpec((haz, half), full),
            pl.BlockSpec((1, half), full),
            pl.BlockSpec((half, out_dim), full),
        ],
        out_specs=pl.BlockSpec((BLK, out_dim), lambda i: (i, 0)),
        out_shape=jax.ShapeDtypeStruct((B, out_dim), jnp.float32),
        compiler_params=pltpu.CompilerParams(dimension_semantics=("parallel",)),
    )(inp, W_embed, W1, b1_2d, W2, b2_2d, W3)


# speedup vs baseline: 1.2601x; 1.2594x over previous
import jax
import jax.numpy as jnp
from jax import lax
from jax.experimental import pallas as pl
from jax.experimental.pallas import tpu as pltpu

BLK = 4096
_PREC = lax.Precision.DEFAULT


def _dot(a, b):
    return jnp.dot(a, b, preferred_element_type=jnp.float32, precision=_PREC)


def _mlp_kernel(inp_ref, we_ref, w1_ref, b1_ref, w2_ref, b2_ref, w3_ref, out_ref):
    x = inp_ref[...]
    e = jnp.maximum(_dot(x, we_ref[...]), 0.0)
    h = jnp.maximum(_dot(e, w1_ref[...]) + b1_ref[...], 0.0)
    h = jnp.maximum(_dot(h, w2_ref[...]) + b2_ref[...], 0.0)
    out_ref[...] = _dot(h, w3_ref[...])


def kernel(inp, W_embed, W1, b1, W2, b2, W3):
    B, inp_dim = inp.shape
    c_embed = W_embed.shape[1]
    haz = W1.shape[1]
    half = W2.shape[1]
    out_dim = W3.shape[1]

    b1_2d = b1.reshape(1, haz)
    b2_2d = b2.reshape(1, half)

    grid = (B // BLK,)
    full = lambda i: (0, 0)
    return pl.pallas_call(
        _mlp_kernel,
        grid=grid,
        in_specs=[
            pl.BlockSpec((BLK, inp_dim), lambda i: (i, 0)),
            pl.BlockSpec((inp_dim, c_embed), full),
            pl.BlockSpec((c_embed, haz), full),
            pl.BlockSpec((1, haz), full),
            pl.BlockSpec((haz, half), full),
            pl.BlockSpec((1, half), full),
            pl.BlockSpec((half, out_dim), full),
        ],
        out_specs=pl.BlockSpec((BLK, out_dim), lambda i: (i, 0)),
        out_shape=jax.ShapeDtypeStruct((B, out_dim), jnp.float32),
        compiler_params=pltpu.CompilerParams(dimension_semantics=("parallel",)),
    )(inp, W_embed, W1, b1_2d, W2, b2_2d, W3)
